# Initial kernel scaffold; baseline (speedup 1.0000x reference)
import jax
import jax.numpy as jnp
from jax.experimental import pallas as pl


def kernel(x, weight):
    raise NotImplementedError("write your pallas kernel here")


# speedup vs baseline: 7.2586x; 7.2586x over previous
import functools

import jax
import jax.numpy as jnp
from jax.experimental import pallas as pl
from jax.experimental.pallas import tpu as pltpu


def _wmax_kernel(x_ref, out_ref, *, w):
    j = pl.program_id(1)

    @pl.when(j == 0)
    def _():
        out_ref[...] = jnp.zeros_like(out_ref)

    v = jnp.abs(x_ref[0])
    core = jnp.max(v[:, 1:-1, :], axis=1)
    e0 = v[:, 0, :]
    eN = v[:, -1, :]
    core2 = jnp.maximum(core[:, :w], core[:, w:])
    r_mid = jnp.maximum(core2, jnp.maximum(e0[:, w:], eN[:, :w]))
    p0 = jnp.maximum(r_mid, e0[:, :w])
    p2 = jnp.maximum(r_mid, eN[:, w:])
    p1 = jnp.maximum(p0, eN[:, w:])

    cols = []
    for p in (p0, p1, p2):
        cfirst = p[:, 0:1]
        clast = p[:, w - 1:w]
        cmid = jnp.max(p[:, 1:w - 1], axis=1, keepdims=True)
        cols.append(jnp.maximum(cfirst, cmid))
        cols.append(jnp.maximum(jnp.maximum(cfirst, cmid), clast))
        cols.append(jnp.maximum(cmid, clast))
    s = jnp.concatenate(cols, axis=1)
    out_ref[0] = jnp.maximum(out_ref[0], s)


def _scale_t_kernel(x_ref, m_ref, o_ref):
    o_ref[0] = jnp.transpose(x_ref[0], (1, 2, 0)) * m_ref[...]


def kernel(x, weight):
    b, c, h, w = x.shape
    o = weight.shape[0]
    k = weight.shape[2]
    sf = 0.5
    kkc = k * k * c

    x = x.astype(jnp.float32)
    weight = weight.astype(jnp.float32)

    hh = h // 2
    x2 = x.reshape(b, c, hh, 2 * w)
    n_par = 2 if b % 2 == 0 else 1
    n_inner = b // n_par

    amax = pl.pallas_call(
        functools.partial(_wmax_kernel, w=w),
        out_shape=jax.ShapeDtypeStruct((n_par, c, 9), jnp.float32),
        grid=(n_par, n_inner),
        in_specs=[pl.BlockSpec((1, c, hh, 2 * w),
                               lambda i, j: (i * n_inner + j, 0, 0, 0))],
        out_specs=pl.BlockSpec((1, c, 9), lambda i, j: (i, 0, 0)),
        compiler_params=pltpu.CompilerParams(
            dimension_semantics=("parallel", "arbitrary")),
    )(x2)

    act_flat = jnp.transpose(jnp.max(amax, axis=0)).reshape(kkc)
    w2 = jnp.transpose(weight, (0, 2, 3, 1)).reshape(o, kkc)
    weight_scale = jnp.max(jnp.abs(w2), axis=0)
    den = weight_scale ** (1.0 - sf)
    scale = (act_flat ** sf) / jnp.where(den == 0.0, 1.0, den)
    scale = jnp.where(den == 0.0, 1.0, scale)
    scale = jnp.where(scale == 0.0, 1.0, scale)
    inv = (1.0 / scale).reshape(k, k, c)
    w_out = (w2 * scale).reshape(o, k, k, c)

    t_row = jnp.stack([inv[0] + inv[1],
                       inv[0] + inv[1] + inv[2],
                       inv[1] + inv[2]])
    m_cls = jnp.stack([t_row[:, 0] + t_row[:, 1],
                       t_row[:, 0] + t_row[:, 1] + t_row[:, 2],
                       t_row[:, 1] + t_row[:, 2]], axis=1)
    ih = jnp.arange(h)
    rci = jnp.where(ih == 0, 0, jnp.where(ih == h - 1, 2, 1))
    iw = jnp.arange(w)
    ccj = jnp.where(iw == 0, 0, jnp.where(iw == w - 1, 2, 1))
    m_full = m_cls[rci[:, None], ccj[None, :]]

    x_out = pl.pallas_call(
        _scale_t_kernel,
        out_shape=jax.ShapeDtypeStruct((b, h, w, c), jnp.float32),
        grid=(b,),
        in_specs=[pl.BlockSpec((1, c, h, w), lambda i: (i, 0, 0, 0)),
                  pl.BlockSpec((h, w, c), lambda i: (0, 0, 0))],
        out_specs=pl.BlockSpec((1, h, w, c), lambda i: (i, 0, 0, 0)),
        compiler_params=pltpu.CompilerParams(
            dimension_semantics=("parallel",)),
    )(x, m_full)

    return w_out, x_out


# speedup vs baseline: 9.2140x; 1.2694x over previous
import functools

import jax
import jax.numpy as jnp
from jax.experimental import pallas as pl
from jax.experimental.pallas import tpu as pltpu


def _wmax_kernel(x_ref, out_ref, *, w):
    j = pl.program_id(1)

    @pl.when(j == 0)
    def _():
        out_ref[...] = jnp.zeros_like(out_ref)

    v = jnp.abs(x_ref[0])
    core = jnp.max(v[:, 1:-1, :], axis=1)
    e0 = v[:, 0, :]
    eN = v[:, -1, :]
    core2 = jnp.maximum(core[:, :w], core[:, w:])
    r_mid = jnp.maximum(core2, jnp.maximum(e0[:, w:], eN[:, :w]))
    p0 = jnp.maximum(r_mid, e0[:, :w])
    p2 = jnp.maximum(r_mid, eN[:, w:])
    p1 = jnp.maximum(p0, eN[:, w:])

    cols = []
    for p in (p0, p1, p2):
        cfirst = p[:, 0:1]
        clast = p[:, w - 1:w]
        cmid = jnp.max(p[:, 1:w - 1], axis=1, keepdims=True)
        cols.append(jnp.maximum(cfirst, cmid))
        cols.append(jnp.maximum(jnp.maximum(cfirst, cmid), clast))
        cols.append(jnp.maximum(cmid, clast))
    s = jnp.concatenate(cols, axis=1)
    out_ref[0] = jnp.maximum(out_ref[0], s)


def _scale_t_kernel(x_ref, m_ref, o_ref):
    o_ref[0] = jnp.transpose(x_ref[0], (1, 2, 0)) * m_ref[...]


def kernel(x, weight):
    b, c, h, w = x.shape
    o = weight.shape[0]
    k = weight.shape[2]
    sf = 0.5
    kkc = k * k * c

    x = x.astype(jnp.float32)
    weight = weight.astype(jnp.float32)

    hh = h // 2
    x2 = x.reshape(b, c, hh, 2 * w)
    n_par = 2 if b % 2 == 0 else 1
    n_inner = b // n_par

    amax = pl.pallas_call(
        functools.partial(_wmax_kernel, w=w),
        out_shape=jax.ShapeDtypeStruct((n_par, c, 9), jnp.float32),
        grid=(n_par, n_inner),
        in_specs=[pl.BlockSpec((1, c, hh, 2 * w),
                               lambda i, j: (i * n_inner + j, 0, 0, 0))],
        out_specs=pl.BlockSpec((1, c, 9), lambda i, j: (i, 0, 0)),
        compiler_params=pltpu.CompilerParams(
            dimension_semantics=("parallel", "arbitrary")),
    )(x2)

    act_flat = jnp.transpose(jnp.max(amax, axis=0)).reshape(kkc)
    w2 = jnp.transpose(weight, (0, 2, 3, 1)).reshape(o, kkc)
    weight_scale = jnp.max(jnp.abs(w2), axis=0)
    den = weight_scale ** (1.0 - sf)
    scale = (act_flat ** sf) / jnp.where(den == 0.0, 1.0, den)
    scale = jnp.where(den == 0.0, 1.0, scale)
    scale = jnp.where(scale == 0.0, 1.0, scale)
    inv = (1.0 / scale).reshape(k, k, c)
    w_out = (w2 * scale).reshape(o, k, k, c)

    t_row = jnp.stack([inv[0] + inv[1],
                       inv[0] + inv[1] + inv[2],
                       inv[1] + inv[2]])
    m_cls = jnp.stack([t_row[:, 0] + t_row[:, 1],
                       t_row[:, 0] + t_row[:, 1] + t_row[:, 2],
                       t_row[:, 1] + t_row[:, 2]], axis=1)
    def _row(a):
        return jnp.concatenate([m_cls[a, 0][None],
                                jnp.broadcast_to(m_cls[a, 1], (w - 2, c)),
                                m_cls[a, 2][None]], axis=0)
    m_full = jnp.concatenate([_row(0)[None],
                              jnp.broadcast_to(_row(1), (h - 2, w, c)),
                              _row(2)[None]], axis=0)

    x_out = pl.pallas_call(
        _scale_t_kernel,
        out_shape=jax.ShapeDtypeStruct((b, h, w, c), jnp.float32),
        grid=(b,),
        in_specs=[pl.BlockSpec((1, c, h, w), lambda i: (i, 0, 0, 0)),
                  pl.BlockSpec((h, w, c), lambda i: (0, 0, 0))],
        out_specs=pl.BlockSpec((1, h, w, c), lambda i: (i, 0, 0, 0)),
        compiler_params=pltpu.CompilerParams(
            dimension_semantics=("parallel",)),
    )(x, m_full)

    return w_out, x_out


# speedup vs baseline: 13.1521x; 1.4274x over previous
import functools

import jax
import jax.numpy as jnp
from jax.experimental import pallas as pl
from jax.experimental.pallas import tpu as pltpu


def _wmax_kernel(x_ref, out_ref, *, w):
    j = pl.program_id(1)

    @pl.when(j == 0)
    def _():
        out_ref[...] = jnp.zeros_like(out_ref)

    v = jnp.abs(x_ref[0])
    h = v.shape[1]
    r_mid = jnp.max(v[:, 1:h - 1, :], axis=1)
    p0 = jnp.maximum(r_mid, v[:, 0, :])
    p2 = jnp.maximum(r_mid, v[:, h - 1, :])
    p1 = jnp.maximum(p0, v[:, h - 1, :])

    cols = []
    for p in (p0, p1, p2):
        cfirst = p[:, 0:1]
        clast = p[:, w - 1:w]
        cmid = jnp.max(p[:, 1:w - 1], axis=1, keepdims=True)
        cols.append(jnp.maximum(cfirst, cmid))
        cols.append(jnp.maximum(jnp.maximum(cfirst, cmid), clast))
        cols.append(jnp.maximum(cmid, clast))
    s = jnp.concatenate(cols, axis=1)
    out_ref[0] = jnp.maximum(out_ref[0], s)


def _scale_t_kernel(x_ref, m_ref, o_ref):
    o_ref[0] = jnp.transpose(x_ref[0], (1, 2, 0)) * m_ref[...]


def kernel(x, weight):
    b, c, h, w = x.shape
    o = weight.shape[0]
    k = weight.shape[2]
    sf = 0.5
    kkc = k * k * c

    x = x.astype(jnp.float32)
    weight = weight.astype(jnp.float32)

    n_par = 2 if b % 2 == 0 else 1
    n_inner = b // n_par

    amax = pl.pallas_call(
        functools.partial(_wmax_kernel, w=w),
        out_shape=jax.ShapeDtypeStruct((n_par, c, 9), jnp.float32),
        grid=(n_par, n_inner),
        in_specs=[pl.BlockSpec((1, c, h, w),
                               lambda i, j: (i * n_inner + j, 0, 0, 0))],
        out_specs=pl.BlockSpec((1, c, 9), lambda i, j: (i, 0, 0)),
        compiler_params=pltpu.CompilerParams(
            dimension_semantics=("parallel", "arbitrary")),
    )(x)

    act_flat = jnp.transpose(jnp.max(amax, axis=0)).reshape(kkc)
    w2 = jnp.transpose(weight, (0, 2, 3, 1)).reshape(o, kkc)
    weight_scale = jnp.max(jnp.abs(w2), axis=0)
    den = weight_scale ** (1.0 - sf)
    scale = (act_flat ** sf) / jnp.where(den == 0.0, 1.0, den)
    scale = jnp.where(den == 0.0, 1.0, scale)
    scale = jnp.where(scale == 0.0, 1.0, scale)
    inv = (1.0 / scale).reshape(k, k, c)
    w_out = (w2 * scale).reshape(o, k, k, c)

    t_row = jnp.stack([inv[0] + inv[1],
                       inv[0] + inv[1] + inv[2],
                       inv[1] + inv[2]])
    m_cls = jnp.stack([t_row[:, 0] + t_row[:, 1],
                       t_row[:, 0] + t_row[:, 1] + t_row[:, 2],
                       t_row[:, 1] + t_row[:, 2]], axis=1)
    def _row(a):
        return jnp.concatenate([m_cls[a, 0][None],
                                jnp.broadcast_to(m_cls[a, 1], (w - 2, c)),
                                m_cls[a, 2][None]], axis=0)
    m_full = jnp.concatenate([_row(0)[None],
                              jnp.broadcast_to(_row(1), (h - 2, w, c)),
                              _row(2)[None]], axis=0)

    x_out = pl.pallas_call(
        _scale_t_kernel,
        out_shape=jax.ShapeDtypeStruct((b, h, w, c), jnp.float32),
        grid=(b,),
        in_specs=[pl.BlockSpec((1, c, h, w), lambda i: (i, 0, 0, 0)),
                  pl.BlockSpec((h, w, c), lambda i: (0, 0, 0))],
        out_specs=pl.BlockSpec((1, h, w, c), lambda i: (i, 0, 0, 0)),
        compiler_params=pltpu.CompilerParams(
            dimension_semantics=("parallel",)),
    )(x, m_full)

    return w_out, x_out


# speedup vs baseline: 13.4825x; 1.0251x over previous
import functools

import jax
import jax.numpy as jnp
from jax.experimental import pallas as pl
from jax.experimental.pallas import tpu as pltpu


def _wmax_kernel(x_ref, out_ref, *, w):
    j = pl.program_id(1)

    @pl.when(j == 0)
    def _():
        out_ref[...] = jnp.zeros_like(out_ref)

    v = jnp.abs(x_ref[0])
    h = v.shape[1]
    r_mid = jnp.max(v[:, 1:h - 1, :], axis=1)
    p0 = jnp.maximum(r_mid, v[:, 0, :])
    p2 = jnp.maximum(r_mid, v[:, h - 1, :])
    p1 = jnp.maximum(p0, v[:, h - 1, :])

    cols = []
    for p in (p0, p1, p2):
        cfirst = p[:, 0:1]
        clast = p[:, w - 1:w]
        cmid = jnp.max(p[:, 1:w - 1], axis=1, keepdims=True)
        cols.append(jnp.maximum(cfirst, cmid))
        cols.append(jnp.maximum(jnp.maximum(cfirst, cmid), clast))
        cols.append(jnp.maximum(cmid, clast))
    s = jnp.concatenate(cols, axis=1)
    out_ref[0] = jnp.maximum(out_ref[0], s)


def _scale_t_kernel(x_ref, mc_ref, o_ref, *, w, c):
    t = jnp.transpose(x_ref[0], (1, 2, 0))
    h = t.shape[0]

    def line(a):
        return jnp.concatenate([mc_ref[a, 0][None],
                                jnp.broadcast_to(mc_ref[a, 1], (w - 2, c)),
                                mc_ref[a, 2][None]], axis=0)

    o_ref[0] = t * line(1)[None]
    o_ref[0, 0] = t[0] * line(0)
    o_ref[0, h - 1] = t[h - 1] * line(2)


def kernel(x, weight):
    b, c, h, w = x.shape
    o = weight.shape[0]
    k = weight.shape[2]
    sf = 0.5
    kkc = k * k * c

    x = x.astype(jnp.float32)
    weight = weight.astype(jnp.float32)

    n_par = 2 if b % 2 == 0 else 1
    n_inner = b // n_par

    amax = pl.pallas_call(
        functools.partial(_wmax_kernel, w=w),
        out_shape=jax.ShapeDtypeStruct((n_par, c, 9), jnp.float32),
        grid=(n_par, n_inner),
        in_specs=[pl.BlockSpec((1, c, h, w),
                               lambda i, j: (i * n_inner + j, 0, 0, 0))],
        out_specs=pl.BlockSpec((1, c, 9), lambda i, j: (i, 0, 0)),
        compiler_params=pltpu.CompilerParams(
            dimension_semantics=("parallel", "arbitrary")),
    )(x)

    act9 = jnp.transpose(jnp.max(amax, axis=0))
    w2 = jnp.transpose(weight, (0, 2, 3, 1)).reshape(o, kkc)
    ws9 = jnp.max(jnp.abs(w2), axis=0).reshape(k * k, c)
    den = ws9 ** (1.0 - sf)
    scale = (act9 ** sf) / jnp.where(den == 0.0, 1.0, den)
    scale = jnp.where(den == 0.0, 1.0, scale)
    scale = jnp.where(scale == 0.0, 1.0, scale)
    inv = (1.0 / scale).reshape(k, k, c)
    w_out = (w2 * scale.reshape(kkc)).reshape(o, k, k, c)

    t_row = jnp.stack([inv[0] + inv[1],
                       inv[0] + inv[1] + inv[2],
                       inv[1] + inv[2]])
    m_cls = jnp.stack([t_row[:, 0] + t_row[:, 1],
                       t_row[:, 0] + t_row[:, 1] + t_row[:, 2],
                       t_row[:, 1] + t_row[:, 2]], axis=1)

    x_out = pl.pallas_call(
        functools.partial(_scale_t_kernel, w=w, c=c),
        out_shape=jax.ShapeDtypeStruct((b, h, w, c), jnp.float32),
        grid=(n_par, n_inner),
        in_specs=[pl.BlockSpec((1, c, h, w),
                               lambda i, j: (i * n_inner + j, 0, 0, 0)),
                  pl.BlockSpec((3, 3, c), lambda i, j: (0, 0, 0))],
        out_specs=pl.BlockSpec((1, h, w, c),
                               lambda i, j: (i * n_inner + j, 0, 0, 0)),
        compiler_params=pltpu.CompilerParams(
            dimension_semantics=("parallel", "arbitrary")),
    )(x, m_cls)

    return w_out, x_out


# speedup vs baseline: 18.1204x; 1.3440x over previous
import functools

import jax
import jax.numpy as jnp
from jax.experimental import pallas as pl
from jax.experimental.pallas import tpu as pltpu


def _wmax_kernel(x_ref, out_ref, *, w):
    j = pl.program_id(1)

    @pl.when(j == 0)
    def _():
        out_ref[...] = jnp.zeros_like(out_ref)

    v = jnp.max(jnp.abs(x_ref[...]), axis=0)
    h = v.shape[1]
    r_mid = jnp.max(v[:, 1:h - 1, :], axis=1)
    p0 = jnp.maximum(r_mid, v[:, 0, :])
    p2 = jnp.maximum(r_mid, v[:, h - 1, :])
    p1 = jnp.maximum(p0, v[:, h - 1, :])

    cols = []
    for p in (p0, p1, p2):
        cfirst = p[:, 0:1]
        clast = p[:, w - 1:w]
        cmid = jnp.max(p[:, 1:w - 1], axis=1, keepdims=True)
        cols.append(jnp.maximum(cfirst, cmid))
        cols.append(jnp.maximum(jnp.maximum(cfirst, cmid), clast))
        cols.append(jnp.maximum(cmid, clast))
    s = jnp.concatenate(cols, axis=1)
    out_ref[0] = jnp.maximum(out_ref[0], s)


def _scale_t_kernel(x_ref, mc_ref, o_ref, *, w, c):
    t = jnp.transpose(x_ref[...], (0, 2, 3, 1))
    h = t.shape[1]

    def line(a):
        return jnp.concatenate([mc_ref[a, 0][None],
                                jnp.broadcast_to(mc_ref[a, 1], (w - 2, c)),
                                mc_ref[a, 2][None]], axis=0)

    o_ref[...] = t * line(1)[None, None]
    o_ref[:, 0] = t[:, 0] * line(0)[None]
    o_ref[:, h - 1] = t[:, h - 1] * line(2)[None]


def kernel(x, weight):
    b, c, h, w = x.shape
    o = weight.shape[0]
    k = weight.shape[2]
    sf = 0.5
    kkc = k * k * c

    x = x.astype(jnp.float32)
    weight = weight.astype(jnp.float32)

    nb = 4 if b % 8 == 0 else (2 if b % 4 == 0 else 1)
    n_par = 2 if (b // nb) % 2 == 0 else 1
    n_inner = b // nb // n_par

    amax = pl.pallas_call(
        functools.partial(_wmax_kernel, w=w),
        out_shape=jax.ShapeDtypeStruct((n_par, c, 9), jnp.float32),
        grid=(n_par, n_inner),
        in_specs=[pl.BlockSpec((nb, c, h, w),
                               lambda i, j: (i * n_inner + j, 0, 0, 0))],
        out_specs=pl.BlockSpec((1, c, 9), lambda i, j: (i, 0, 0)),
        compiler_params=pltpu.CompilerParams(
            dimension_semantics=("parallel", "arbitrary")),
    )(x)

    act9 = jnp.transpose(jnp.max(amax, axis=0))
    w2 = jnp.transpose(weight, (0, 2, 3, 1)).reshape(o, kkc)
    ws9 = jnp.max(jnp.abs(w2), axis=0).reshape(k * k, c)
    den = ws9 ** (1.0 - sf)
    scale = (act9 ** sf) / jnp.where(den == 0.0, 1.0, den)
    scale = jnp.where(den == 0.0, 1.0, scale)
    scale = jnp.where(scale == 0.0, 1.0, scale)
    inv = (1.0 / scale).reshape(k, k, c)
    w_out = (w2 * scale.reshape(kkc)).reshape(o, k, k, c)

    t_row = jnp.stack([inv[0] + inv[1],
                       inv[0] + inv[1] + inv[2],
                       inv[1] + inv[2]])
    m_cls = jnp.stack([t_row[:, 0] + t_row[:, 1],
                       t_row[:, 0] + t_row[:, 1] + t_row[:, 2],
                       t_row[:, 1] + t_row[:, 2]], axis=1)

    x_out = pl.pallas_call(
        functools.partial(_scale_t_kernel, w=w, c=c),
        out_shape=jax.ShapeDtypeStruct((b, h, w, c), jnp.float32),
        grid=(n_par, n_inner),
        in_specs=[pl.BlockSpec((nb, c, h, w),
                               lambda i, j: (i * n_inner + j, 0, 0, 0)),
                  pl.BlockSpec((3, 3, c), lambda i, j: (0, 0, 0))],
        out_specs=pl.BlockSpec((nb, h, w, c),
                               lambda i, j: (i * n_inner + j, 0, 0, 0)),
        compiler_params=pltpu.CompilerParams(
            dimension_semantics=("parallel", "arbitrary")),
    )(x, m_cls)

    return w_out, x_out


# speedup vs baseline: 19.7036x; 1.0874x over previous
import functools

import jax
import jax.numpy as jnp
from jax.experimental import pallas as pl
from jax.experimental.pallas import tpu as pltpu


def _wmax_kernel(x_ref, out_ref, *, w):
    j = pl.program_id(1)

    @pl.when(j == 0)
    def _():
        out_ref[...] = jnp.zeros_like(out_ref)

    v = jnp.max(jnp.abs(x_ref[...]), axis=0)
    h = v.shape[1]
    r_mid = jnp.max(v[:, 1:h - 1, :], axis=1)
    p0 = jnp.maximum(r_mid, v[:, 0, :])
    p2 = jnp.maximum(r_mid, v[:, h - 1, :])
    p1 = jnp.maximum(p0, v[:, h - 1, :])

    cols = []
    for p in (p0, p1, p2):
        cfirst = p[:, 0:1]
        clast = p[:, w - 1:w]
        cmid = jnp.max(p[:, 1:w - 1], axis=1, keepdims=True)
        cols.append(jnp.maximum(cfirst, cmid))
        cols.append(jnp.maximum(jnp.maximum(cfirst, cmid), clast))
        cols.append(jnp.maximum(cmid, clast))
    s = jnp.concatenate(cols, axis=1)
    out_ref[0] = jnp.maximum(out_ref[0], s)


def _scale_t_kernel(x_ref, mc_ref, o_ref, *, w, c):
    nb, _, h, _ = x_ref.shape
    a = jnp.transpose(x_ref[...], (0, 2, 1, 3))
    t = jnp.transpose(a, (0, 1, 3, 2))

    def line(a):
        return jnp.concatenate([mc_ref[a, 0][None],
                                jnp.broadcast_to(mc_ref[a, 1], (w - 2, c)),
                                mc_ref[a, 2][None]], axis=0)

    o_ref[...] = t * line(1)[None, None]
    o_ref[:, 0] = t[:, 0] * line(0)[None]
    o_ref[:, h - 1] = t[:, h - 1] * line(2)[None]


def kernel(x, weight):
    b, c, h, w = x.shape
    o = weight.shape[0]
    k = weight.shape[2]
    sf = 0.5
    kkc = k * k * c

    x = x.astype(jnp.float32)
    weight = weight.astype(jnp.float32)

    nb = 4 if b % 8 == 0 else (2 if b % 4 == 0 else 1)
    n_par = 2 if (b // nb) % 2 == 0 else 1
    n_inner = b // nb // n_par

    amax = pl.pallas_call(
        functools.partial(_wmax_kernel, w=w),
        out_shape=jax.ShapeDtypeStruct((n_par, c, 9), jnp.float32),
        grid=(n_par, n_inner),
        in_specs=[pl.BlockSpec((nb, c, h, w),
                               lambda i, j: (i * n_inner + j, 0, 0, 0))],
        out_specs=pl.BlockSpec((1, c, 9), lambda i, j: (i, 0, 0)),
        compiler_params=pltpu.CompilerParams(
            dimension_semantics=("parallel", "arbitrary")),
    )(x)

    act9 = jnp.transpose(jnp.max(amax, axis=0))
    w2 = jnp.transpose(weight, (0, 2, 3, 1)).reshape(o, kkc)
    ws9 = jnp.max(jnp.abs(w2), axis=0).reshape(k * k, c)
    den = ws9 ** (1.0 - sf)
    scale = (act9 ** sf) / jnp.where(den == 0.0, 1.0, den)
    scale = jnp.where(den == 0.0, 1.0, scale)
    scale = jnp.where(scale == 0.0, 1.0, scale)
    inv = (1.0 / scale).reshape(k, k, c)
    w_out = (w2 * scale.reshape(kkc)).reshape(o, k, k, c)

    t_row = jnp.stack([inv[0] + inv[1],
                       inv[0] + inv[1] + inv[2],
                       inv[1] + inv[2]])
    m_cls = jnp.stack([t_row[:, 0] + t_row[:, 1],
                       t_row[:, 0] + t_row[:, 1] + t_row[:, 2],
                       t_row[:, 1] + t_row[:, 2]], axis=1)

    x_out = pl.pallas_call(
        functools.partial(_scale_t_kernel, w=w, c=c),
        out_shape=jax.ShapeDtypeStruct((b, h, w, c), jnp.float32),
        grid=(n_par, n_inner),
        in_specs=[pl.BlockSpec((nb, c, h, w),
                               lambda i, j: (i * n_inner + j, 0, 0, 0)),
                  pl.BlockSpec((3, 3, c), lambda i, j: (0, 0, 0))],
        out_specs=pl.BlockSpec((nb, h, w, c),
                               lambda i, j: (i * n_inner + j, 0, 0, 0)),
        compiler_params=pltpu.CompilerParams(
            dimension_semantics=("parallel", "arbitrary")),
    )(x, m_cls)

    return w_out, x_out


# speedup vs baseline: 19.9997x; 1.0150x over previous
import functools

import jax
import jax.numpy as jnp
from jax.experimental import pallas as pl
from jax.experimental.pallas import tpu as pltpu


def _wmax_kernel(x_ref, out_ref, *, w):
    j = pl.program_id(1)

    @pl.when(j == 0)
    def _():
        out_ref[...] = jnp.zeros_like(out_ref)

    v = jnp.max(jnp.abs(x_ref[...]), axis=0)
    h = v.shape[1]
    r_mid = jnp.max(v[:, 1:h - 1, :], axis=1)
    p0 = jnp.maximum(r_mid, v[:, 0, :])
    p2 = jnp.maximum(r_mid, v[:, h - 1, :])
    p1 = jnp.maximum(p0, v[:, h - 1, :])

    cols = []
    for p in (p0, p1, p2):
        cfirst = p[:, 0:1]
        clast = p[:, w - 1:w]
        cmid = jnp.max(p[:, 1:w - 1], axis=1, keepdims=True)
        cols.append(jnp.maximum(cfirst, cmid))
        cols.append(jnp.maximum(jnp.maximum(cfirst, cmid), clast))
        cols.append(jnp.maximum(cmid, clast))
    s = jnp.concatenate(cols, axis=1)
    out_ref[0] = jnp.maximum(out_ref[0], s)


def _scale_t_kernel(x_ref, amax_ref, ws_ref, o_ref, lines_ref, *, w, c):
    nb, _, h, _ = x_ref.shape

    @pl.when(pl.program_id(1) == 0)
    def _():
        act = jnp.max(amax_ref[...], axis=0)
        den = jnp.sqrt(ws_ref[...])
        scale = jnp.sqrt(act) / jnp.where(den == 0.0, 1.0, den)
        scale = jnp.where(den == 0.0, 1.0, scale)
        scale = jnp.where(scale == 0.0, 1.0, scale)
        inv = 1.0 / scale

        tr0 = inv[:, 0:3] + inv[:, 3:6]
        tr1 = tr0 + inv[:, 6:9]
        tr2 = inv[:, 3:6] + inv[:, 6:9]

        def line(tr):
            e0 = jnp.transpose(tr[:, 0:1] + tr[:, 1:2])
            e1 = jnp.transpose(tr[:, 0:1] + tr[:, 1:2] + tr[:, 2:3])
            e2 = jnp.transpose(tr[:, 1:2] + tr[:, 2:3])
            return jnp.concatenate([e0, jnp.broadcast_to(e1, (w - 2, c)),
                                    e2], axis=0)

        lines_ref[0] = line(tr0)
        lines_ref[1] = line(tr1)
        lines_ref[2] = line(tr2)

    a = jnp.transpose(x_ref[...], (0, 2, 1, 3))
    t = jnp.transpose(a, (0, 1, 3, 2))
    o_ref[...] = t * lines_ref[1][None, None]
    o_ref[:, 0] = t[:, 0] * lines_ref[0][None]
    o_ref[:, h - 1] = t[:, h - 1] * lines_ref[2][None]


def kernel(x, weight):
    b, c, h, w = x.shape
    o = weight.shape[0]
    k = weight.shape[2]
    sf = 0.5
    kkc = k * k * c

    x = x.astype(jnp.float32)
    weight = weight.astype(jnp.float32)

    nb = 4 if b % 8 == 0 else (2 if b % 4 == 0 else 1)
    n_par = 2 if (b // nb) % 2 == 0 else 1
    n_inner = b // nb // n_par

    amax = pl.pallas_call(
        functools.partial(_wmax_kernel, w=w),
        out_shape=jax.ShapeDtypeStruct((n_par, c, 9), jnp.float32),
        grid=(n_par, n_inner),
        in_specs=[pl.BlockSpec((nb, c, h, w),
                               lambda i, j: (i * n_inner + j, 0, 0, 0))],
        out_specs=pl.BlockSpec((1, c, 9), lambda i, j: (i, 0, 0)),
        compiler_params=pltpu.CompilerParams(
            dimension_semantics=("parallel", "arbitrary")),
    )(x)

    act9 = jnp.transpose(jnp.max(amax, axis=0))
    w2 = jnp.transpose(weight, (0, 2, 3, 1)).reshape(o, kkc)
    weight_scale = jnp.max(jnp.abs(w2), axis=0)
    ws9 = weight_scale.reshape(k * k, c)
    den = ws9 ** (1.0 - sf)
    scale = (act9 ** sf) / jnp.where(den == 0.0, 1.0, den)
    scale = jnp.where(den == 0.0, 1.0, scale)
    scale = jnp.where(scale == 0.0, 1.0, scale)
    w_out = (w2 * scale.reshape(kkc)).reshape(o, k, k, c)

    ws_c9 = jnp.transpose(ws9)
    x_out = pl.pallas_call(
        functools.partial(_scale_t_kernel, w=w, c=c),
        out_shape=jax.ShapeDtypeStruct((b, h, w, c), jnp.float32),
        grid=(n_par, n_inner),
        in_specs=[pl.BlockSpec((nb, c, h, w),
                               lambda i, j: (i * n_inner + j, 0, 0, 0)),
                  pl.BlockSpec((n_par, c, 9), lambda i, j: (0, 0, 0)),
                  pl.BlockSpec((c, 9), lambda i, j: (0, 0))],
        out_specs=pl.BlockSpec((nb, h, w, c),
                               lambda i, j: (i * n_inner + j, 0, 0, 0)),
        scratch_shapes=[pltpu.VMEM((3, w, c), jnp.float32)],
        compiler_params=pltpu.CompilerParams(
            dimension_semantics=("parallel", "arbitrary")),
    )(x, amax, ws_c9)

    return w_out, x_out


# speedup vs baseline: 20.4051x; 1.0203x over previous
import functools

import jax
import jax.numpy as jnp
from jax.experimental import pallas as pl
from jax.experimental.pallas import tpu as pltpu


def _fused_kernel(x_ref, ws_ref, o_ref, amax_ref, acc_ref, lines_ref, *, w, c):
    p = pl.program_id(0)
    j = pl.program_id(1)
    nj = pl.num_programs(1)
    nb, _, h, _ = x_ref.shape

    @pl.when(p == 0)
    def _():
        @pl.when(j == 0)
        def _():
            acc_ref[...] = jnp.zeros_like(acc_ref)

        v = jnp.max(jnp.abs(x_ref[...]), axis=0)
        r_mid = jnp.max(v[:, 1:h - 1, :], axis=1)
        p0 = jnp.maximum(r_mid, v[:, 0, :])
        p2 = jnp.maximum(r_mid, v[:, h - 1, :])
        p1 = jnp.maximum(p0, v[:, h - 1, :])

        cols = []
        for pp in (p0, p1, p2):
            cfirst = pp[:, 0:1]
            clast = pp[:, w - 1:w]
            cmid = jnp.max(pp[:, 1:w - 1], axis=1, keepdims=True)
            cols.append(jnp.maximum(cfirst, cmid))
            cols.append(jnp.maximum(jnp.maximum(cfirst, cmid), clast))
            cols.append(jnp.maximum(cmid, clast))
        s = jnp.concatenate(cols, axis=1)
        acc_ref[...] = jnp.maximum(acc_ref[...], s)

        @pl.when(j == nj - 1)
        def _():
            amax_ref[...] = acc_ref[...]

    @pl.when((p == 1) & (j == 0))
    def _():
        act = acc_ref[...]
        den = jnp.sqrt(ws_ref[...])
        scale = jnp.sqrt(act) / jnp.where(den == 0.0, 1.0, den)
        scale = jnp.where(den == 0.0, 1.0, scale)
        scale = jnp.where(scale == 0.0, 1.0, scale)
        inv = 1.0 / scale

        tr0 = inv[:, 0:3] + inv[:, 3:6]
        tr1 = tr0 + inv[:, 6:9]
        tr2 = inv[:, 3:6] + inv[:, 6:9]

        def line(tr):
            e0 = jnp.transpose(tr[:, 0:1] + tr[:, 1:2])
            e1 = jnp.transpose(tr[:, 0:1] + tr[:, 1:2] + tr[:, 2:3])
            e2 = jnp.transpose(tr[:, 1:2] + tr[:, 2:3])
            return jnp.concatenate([e0, jnp.broadcast_to(e1, (w - 2, c)),
                                    e2], axis=0)

        lines_ref[0] = line(tr0)
        lines_ref[1] = line(tr1)
        lines_ref[2] = line(tr2)

    @pl.when(p == 1)
    def _():
        a = jnp.transpose(x_ref[...], (0, 2, 1, 3))
        t = jnp.transpose(a, (0, 1, 3, 2))
        o_ref[...] = t * lines_ref[1][None, None]
        o_ref[:, 0] = t[:, 0] * lines_ref[0][None]
        o_ref[:, h - 1] = t[:, h - 1] * lines_ref[2][None]


def kernel(x, weight):
    b, c, h, w = x.shape
    o = weight.shape[0]
    k = weight.shape[2]
    sf = 0.5
    kkc = k * k * c

    x = x.astype(jnp.float32)
    weight = weight.astype(jnp.float32)

    w2 = jnp.transpose(weight, (0, 2, 3, 1)).reshape(o, kkc)
    ws9 = jnp.max(jnp.abs(w2), axis=0).reshape(k * k, c)
    ws_c9 = jnp.transpose(ws9)

    nb = 4 if b % 8 == 0 else (2 if b % 4 == 0 else 1)
    nj = b // nb

    x_out, amax = pl.pallas_call(
        functools.partial(_fused_kernel, w=w, c=c),
        out_shape=(jax.ShapeDtypeStruct((b, h, w, c), jnp.float32),
                   jax.ShapeDtypeStruct((c, 9), jnp.float32)),
        grid=(2, nj),
        in_specs=[pl.BlockSpec((nb, c, h, w), lambda p, j: (j, 0, 0, 0)),
                  pl.BlockSpec((c, 9), lambda p, j: (0, 0))],
        out_specs=(pl.BlockSpec((nb, h, w, c), lambda p, j: (p * j, 0, 0, 0)),
                   pl.BlockSpec((c, 9), lambda p, j: (0, 0))),
        scratch_shapes=[pltpu.VMEM((c, 9), jnp.float32),
                        pltpu.VMEM((3, w, c), jnp.float32)],
        compiler_params=pltpu.CompilerParams(
            dimension_semantics=("arbitrary", "arbitrary")),
    )(x, ws_c9)

    act9 = jnp.transpose(amax)
    den = ws9 ** (1.0 - sf)
    scale = (act9 ** sf) / jnp.where(den == 0.0, 1.0, den)
    scale = jnp.where(den == 0.0, 1.0, scale)
    scale = jnp.where(scale == 0.0, 1.0, scale)
    w_out = (w2 * scale.reshape(kkc)).reshape(o, k, k, c)

    return w_out, x_out
